# baseline (speedup 1.0000x reference)
import functools

import jax
import jax.numpy as jnp
import numpy as _np
from jax import lax
from jax.experimental import pallas as pl
from jax.experimental.pallas import tpu as pltpu
from jax.experimental.pallas import tpu_sc as plsc

H = 250
W = 400
T = 64
U0 = 0.15
D = 0.05 * 20
F = 0.5 * 20
FPAR = 0.15

NF = 32

_CD = [float(_np.exp(-(2.0 ** k) / D)) for k in range(6)]
_CF = [float(_np.exp(-(2.0 ** k) / F)) for k in range(6)]


def _make_sc_kernel():
    mesh = plsc.VectorSubcoreMesh(core_axis_name="c", subcore_axis_name="s")

    @functools.partial(
        pl.kernel,
        mesh=mesh,
        out_type=[
            jax.ShapeDtypeStruct((NF, H, W), jnp.float32),
            jax.ShapeDtypeStruct((NF, H, W), jnp.float32),
        ],
        scratch_types=[
            pltpu.VMEM((T, 1, W), jnp.float32),
            pltpu.VMEM((33, 1, W), jnp.int32),
            pltpu.VMEM((NF, 1, W), jnp.float32),
            pltpu.VMEM((NF, 1, W), jnp.float32),
        ],
    )
    def sc_kernel(sp_hbm, outu_hbm, outR_hbm, spike_v, ng_v, outu_v, outR_v):
        nc = 2
        wid = lax.axis_index("s") * nc + lax.axis_index("c")
        nrows = jnp.where(wid < 26, 8, 7)
        row0 = 8 * wid - jnp.maximum(wid - 26, 0)

        def run_groups(bases):
            nb = len(bases)
            lanes = [pl.ds(b, 16) for b in bases]

            nxt = [jnp.full((16,), 256, jnp.int32) for _ in range(nb)]
            for t in range(63, 32, -1):
                for k in range(nb):
                    s = spike_v[t, 0, lanes[k]]
                    nxt[k] = jnp.where(s != 0.0, t, nxt[k])
            for t in range(32, 0, -1):
                for k in range(nb):
                    s = spike_v[t, 0, lanes[k]]
                    nxt[k] = jnp.where(s != 0.0, t, nxt[k])
                    ng_v[t, 0, lanes[k]] = nxt[k]

            last, Rst, ust = [], [], []
            for k in range(nb):
                s0 = spike_v[0, 0, lanes[k]]
                last.append(jnp.where(s0 != 0.0, 0, -128))
                Rst.append(jnp.full((16,), 1.0, jnp.float32))
                ust.append(jnp.full((16,), U0, jnp.float32))
            for t in range(1, 33):
                for k in range(nb):
                    s = spike_v[t, 0, lanes[k]]
                    nx = ng_v[t, 0, lanes[k]]
                    s_zero = s == 0.0
                    isi = nx - last[k]
                    mask = (isi < 64) & (s_zero | (isi == 1))
                    eD = jnp.where((isi & 1) != 0, _CD[0], 1.0)
                    eF = jnp.where((isi & 1) != 0, _CF[0], 1.0)
                    for b in range(1, 6):
                        bit = (isi & (1 << b)) != 0
                        eD = eD * jnp.where(bit, _CD[b], 1.0)
                        eF = eF * jnp.where(bit, _CF[b], 1.0)
                    Rn = 1.0 - (1.0 - Rst[k] * (1.0 - ust[k])) * eD
                    un = U0 + (ust[k] + FPAR * (1.0 - ust[k]) - U0) * eF
                    Rst[k] = jnp.where(mask, Rn, Rst[k])
                    ust[k] = jnp.where(mask, un, ust[k])
                    outu_v[t - 1, 0, lanes[k]] = ust[k]
                    outR_v[t - 1, 0, lanes[k]] = Rst[k]
                    last[k] = jnp.where(s_zero, last[k], t)
            return None

        def do_row(r, carry):
            row = row0 + r
            pltpu.sync_copy(sp_hbm.at[0, :, pl.ds(row, 1), :], spike_v)

            def do_quint(g, carry2):
                b = g * 80
                run_groups([b, b + 16, b + 32, b + 48, b + 64])
                return carry2

            lax.fori_loop(0, 5, do_quint, 0)

            pltpu.sync_copy(outu_v, outu_hbm.at[:, pl.ds(row, 1), :])
            pltpu.sync_copy(outR_v, outR_hbm.at[:, pl.ds(row, 1), :])
            return carry

        lax.fori_loop(0, nrows, do_row, 0)

    return sc_kernel


_sc_kernel = _make_sc_kernel()


def _tc_body(u_ref, R_ref, out_ref):
    t = pl.program_id(0)

    @pl.when((t >= 1) & (t <= 32))
    def _():
        u = u_ref[0]
        R = R_ref[0]
        au = (u - U0) / (F - U0 + u * (1.0 - FPAR))
        aR = (1.0 - R) / (1.0 - R * (1.0 - u))
        image = -1.0 / (F * jnp.log(au)) + -1.0 / (D * jnp.log(aR))
        mn = jnp.min(image)
        mx = jnp.max(image)
        out_ref[0] = jnp.where(mx != mn, (image - mn) / (mx - mn), image)

    @pl.when((t == 0) | (t > 32))
    def _():
        out_ref[0] = jnp.zeros((H, W), jnp.float32)


_tc_norm = pl.pallas_call(
    _tc_body,
    grid=(T,),
    in_specs=[
        pl.BlockSpec((1, H, W), lambda t: (jnp.clip(t - 1, 0, NF - 1), 0, 0)),
        pl.BlockSpec((1, H, W), lambda t: (jnp.clip(t - 1, 0, NF - 1), 0, 0)),
    ],
    out_specs=pl.BlockSpec((1, H, W), lambda t: (t, 0, 0)),
    out_shape=jax.ShapeDtypeStruct((T, H, W), jnp.float32),
)


def kernel(spikes):
    u_frames, R_frames = _sc_kernel(spikes)
    return _tc_norm(u_frames, R_frames)


# speedup vs baseline: 2.0337x; 2.0337x over previous
import functools

import jax
import jax.numpy as jnp
from jax import lax
from jax.experimental import pallas as pl
from jax.experimental.pallas import tpu as pltpu
from jax.experimental.pallas import tpu_sc as plsc

H = 250
W = 400
T = 64
U0 = 0.15
D = 0.05 * 20
F = 0.5 * 20
FPAR = 0.15

NF = 32


def _take16(vec, idx):
    return lax.gather(
        vec,
        idx.reshape(16, 1),
        lax.GatherDimensionNumbers(
            offset_dims=(), collapsed_slice_dims=(0,), start_index_map=(0,)),
        slice_sizes=(1,),
        mode=lax.GatherScatterMode.PROMISE_IN_BOUNDS)


def _pack_body(sp_ref, out_ref):
    s = sp_ref[0]
    acc = jnp.zeros((H, W), jnp.int32)
    for j in range(32):
        bit = jnp.int32(-2147483648) if j == 31 else jnp.int32(1 << j)
        acc = acc | jnp.where(s[j] != 0.0, bit, 0)
    out_ref[0] = acc


_tc_pack = pl.pallas_call(
    _pack_body,
    grid=(2,),
    in_specs=[pl.BlockSpec((1, 32, H, W), lambda w: (0, w, 0, 0))],
    out_specs=pl.BlockSpec((1, H, W), lambda w: (w, 0, 0)),
    out_shape=jax.ShapeDtypeStruct((2, H, W), jnp.int32),
)


def _make_sc_kernel():
    mesh = plsc.VectorSubcoreMesh(core_axis_name="c", subcore_axis_name="s")

    @functools.partial(
        pl.kernel,
        mesh=mesh,
        out_type=[
            jax.ShapeDtypeStruct((NF, H, W), jnp.float32),
            jax.ShapeDtypeStruct((NF, H, W), jnp.float32),
        ],
        scratch_types=[
            pltpu.VMEM((2, 1, W), jnp.int32),
            pltpu.VMEM((NF, 1, W), jnp.float32),
            pltpu.VMEM((NF, 1, W), jnp.float32),
            pltpu.VMEM((32,), jnp.float32),
            pltpu.VMEM((64,), jnp.float32),
        ],
    )
    def sc_kernel(pk_hbm, lutD_hbm, lutF_hbm, outu_hbm, outR_hbm,
                  wbuf_v, outu_v, outR_v, lutD_v, lutF_v):
        nc = 2
        wid = lax.axis_index("s") * nc + lax.axis_index("c")
        nrows = jnp.where(wid < 26, 8, 7)
        row0 = 8 * wid - jnp.maximum(wid - 26, 0)

        pltpu.sync_copy(lutD_hbm, lutD_v)
        pltpu.sync_copy(lutF_hbm, lutF_v)
        lutD = [lutD_v[pl.ds(16 * j, 16)] for j in range(2)]
        lutF = [lutF_v[pl.ds(16 * j, 16)] for j in range(4)]

        def run_groups(bases):
            nb = len(bases)
            out_lanes = [pl.ds(b, 16) for b in bases]
            rlo = [wbuf_v[0, 0, out_lanes[k]].astype(jnp.uint32)
                   for k in range(nb)]
            rhi = [wbuf_v[1, 0, out_lanes[k]].astype(jnp.uint32)
                   for k in range(nb)]
            last, Rst, ust = [], [], []
            for k in range(nb):
                s0 = (rlo[k] & 1) != 0
                last.append(jnp.where(s0, 0, -128))
                Rst.append(jnp.full((16,), 1.0, jnp.float32))
                ust.append(jnp.full((16,), U0, jnp.float32))
            for t in range(1, 33):
                for k in range(nb):
                    rlo[k] = (rlo[k] >> 1) | (rhi[k] << 31)
                    rhi[k] = rhi[k] >> 1
                    s_zero = (rlo[k] & 1) == 0
                    lo_nz = rlo[k] != 0
                    xsel = jnp.where(lo_nz, rlo[k], rhi[k])
                    zr = xsel == 0
                    lsb = xsel & (jnp.uint32(0) - xsel)
                    f = lsb.astype(jnp.float32)
                    e = lax.bitcast_convert_type(f, jnp.int32) >> 23
                    adj = jnp.where(lo_nz, -127, -95)
                    isi = (t - last[k]) + e + adj
                    isi = jnp.where(zr, 999, isi)
                    mask = (isi < 64) & (s_zero | (isi == 1))
                    idx = jnp.minimum(isi, 63)
                    idx_c = jnp.minimum(isi, 31)
                    lo_c = idx_c & 15
                    is_lo = idx < 16
                    eD = jnp.where(is_lo,
                                   _take16(lutD[0], lo_c),
                                   _take16(lutD[1], lo_c))
                    lo = idx & 15
                    f01 = jnp.where(is_lo,
                                    _take16(lutF[0], lo),
                                    _take16(lutF[1], lo))
                    f23 = jnp.where(idx < 48,
                                    _take16(lutF[2], lo),
                                    _take16(lutF[3], lo))
                    eF = jnp.where(idx < 32, f01, f23)
                    Rn = 1.0 - (1.0 - Rst[k] * (1.0 - ust[k])) * eD
                    un = U0 + (ust[k] + FPAR * (1.0 - ust[k]) - U0) * eF
                    Rst[k] = jnp.where(mask, Rn, Rst[k])
                    ust[k] = jnp.where(mask, un, ust[k])
                    outu_v[t - 1, 0, out_lanes[k]] = ust[k]
                    outR_v[t - 1, 0, out_lanes[k]] = Rst[k]
                    last[k] = jnp.where(s_zero, last[k], t)
            return None

        def do_row(r, carry):
            row = row0 + r
            pltpu.sync_copy(pk_hbm.at[:, pl.ds(row, 1), :], wbuf_v)

            def do_pair(g, carry2):
                b = g * 32
                run_groups([b, b + 16])
                return carry2

            lax.fori_loop(0, 12, do_pair, 0)
            run_groups([384])
            pltpu.sync_copy(outu_v, outu_hbm.at[:, pl.ds(row, 1), :])
            pltpu.sync_copy(outR_v, outR_hbm.at[:, pl.ds(row, 1), :])
            return carry

        lax.fori_loop(0, nrows, do_row, 0)

    return sc_kernel


_sc_kernel = _make_sc_kernel()


def _tc_body(u_ref, R_ref, out_ref):
    t = pl.program_id(0)

    @pl.when((t >= 1) & (t <= 32))
    def _():
        u = u_ref[0]
        R = R_ref[0]
        au = (u - U0) / (F - U0 + u * (1.0 - FPAR))
        aR = (1.0 - R) / (1.0 - R * (1.0 - u))
        image = -1.0 / (F * jnp.log(au)) + -1.0 / (D * jnp.log(aR))
        mn = jnp.min(image)
        mx = jnp.max(image)
        out_ref[0] = jnp.where(mx != mn, (image - mn) / (mx - mn), image)

    @pl.when((t == 0) | (t > 32))
    def _():
        out_ref[0] = jnp.zeros((H, W), jnp.float32)


_tc_norm = pl.pallas_call(
    _tc_body,
    grid=(T,),
    in_specs=[
        pl.BlockSpec((1, H, W), lambda t: (jnp.clip(t - 1, 0, NF - 1), 0, 0)),
        pl.BlockSpec((1, H, W), lambda t: (jnp.clip(t - 1, 0, NF - 1), 0, 0)),
    ],
    out_specs=pl.BlockSpec((1, H, W), lambda t: (t, 0, 0)),
    out_shape=jax.ShapeDtypeStruct((T, H, W), jnp.float32),
)


def kernel(spikes):
    i = jnp.arange(64, dtype=jnp.float32)
    lutD = jnp.exp(-i / D)[:32]
    lutF = jnp.exp(-i / F)
    packed = _tc_pack(spikes)
    u_frames, R_frames = _sc_kernel(packed, lutD, lutF)
    return _tc_norm(u_frames, R_frames)
